# baseline (speedup 1.0000x reference)
import jax
import jax.numpy as jnp
from jax.experimental import pallas as pl
from jax.experimental.pallas import tpu as pltpu

B, N, D, E = 16, 512, 32, 4


def _ggnn_kernel(x_ref, edges_ref, We_ref, Wz_ref, Uz_ref, bz_ref,
                 Wr_ref, Ur_ref, br_ref, Wh_ref, Uh_ref, bh_ref,
                 Wo_ref, bo_ref, out_ref, m_acc):
    e = pl.program_id(1)
    x = x_ref[0]

    tx = jnp.dot(x, We_ref[0], preferred_element_type=jnp.float32)
    contrib = jnp.dot(edges_ref[0, 0].astype(jnp.bfloat16),
                      tx.astype(jnp.bfloat16),
                      preferred_element_type=jnp.float32)

    @pl.when(e == 0)
    def _init():
        m_acc[...] = contrib

    @pl.when(e != 0)
    def _accum():
        m_acc[...] += contrib

    @pl.when(e == E - 1)
    def _epilogue():
        m = m_acc[...]
        z = jax.nn.sigmoid(jnp.dot(m, Wz_ref[...]) + jnp.dot(x, Uz_ref[...])
                           + bz_ref[...])
        r = jax.nn.sigmoid(jnp.dot(m, Wr_ref[...]) + jnp.dot(x, Ur_ref[...])
                           + br_ref[...])
        h_til = jnp.tanh(jnp.dot(m, Wh_ref[...])
                         + jnp.dot(r * x, Uh_ref[...]) + bh_ref[...])
        h = (1.0 - z) * x + z * h_til

        logits = jnp.dot(h, Wo_ref[...]) + bo_ref[...]
        lmax = jnp.max(logits, axis=1, keepdims=True)
        shifted = logits - lmax
        lse = jnp.log(jnp.sum(jnp.exp(shifted), axis=1, keepdims=True))
        out_ref[0] = shifted - lse


@jax.jit
def kernel(x_padded, x_lengths, edges, fingers, W_edge, Wz, Uz, bz,
           Wr, Ur, br, Wh, Uh, bh, W_out, b_out):
    del x_lengths, fingers
    grid = (B, E)

    full = lambda b, e: (0, 0)
    out = pl.pallas_call(
        _ggnn_kernel,
        grid=grid,
        in_specs=[
            pl.BlockSpec((1, N, D), lambda b, e: (b, 0, 0)),
            pl.BlockSpec((1, 1, N, N), lambda b, e: (b, e, 0, 0)),
            pl.BlockSpec((1, D, D), lambda b, e: (e, 0, 0)),
            pl.BlockSpec((D, D), full),
            pl.BlockSpec((D, D), full),
            pl.BlockSpec((1, D), full),
            pl.BlockSpec((D, D), full),
            pl.BlockSpec((D, D), full),
            pl.BlockSpec((1, D), full),
            pl.BlockSpec((D, D), full),
            pl.BlockSpec((D, D), full),
            pl.BlockSpec((1, D), full),
            pl.BlockSpec((D, 5), full),
            pl.BlockSpec((1, 5), full),
        ],
        out_specs=pl.BlockSpec((1, N, 5), lambda b, e: (b, 0, 0)),
        out_shape=jax.ShapeDtypeStruct((B, N, 5), jnp.float32),
        scratch_shapes=[pltpu.VMEM((N, D), jnp.float32)],
        compiler_params=pltpu.CompilerParams(
            dimension_semantics=("parallel", "arbitrary")),
    )(x_padded, edges, W_edge, Wz, Uz, bz.reshape(1, D),
      Wr, Ur, br.reshape(1, D), Wh, Uh, bh.reshape(1, D),
      W_out, b_out.reshape(1, 5))
    return out


# speedup vs baseline: 1.6725x; 1.6725x over previous
import jax
import jax.numpy as jnp
from jax.experimental import pallas as pl

B, N, D, E = 16, 512, 32, 4


def _ggnn_kernel(xT_ref, edges_ref, WeT_ref, WzT_ref, UzT_ref, bzT_ref,
                 WrT_ref, UrT_ref, brT_ref, WhT_ref, UhT_ref, bhT_ref,
                 WoT_ref, boT_ref, out_ref):
    xT = xT_ref[0]

    mT = jnp.zeros((D, N), dtype=jnp.float32)
    for e in range(E):
        txT = jnp.dot(WeT_ref[e], xT, preferred_element_type=jnp.float32)
        mT = mT + jax.lax.dot_general(
            txT.astype(jnp.bfloat16), edges_ref[0, e].astype(jnp.bfloat16),
            dimension_numbers=(((1,), (1,)), ((), ())),
            preferred_element_type=jnp.float32)

    z = jax.nn.sigmoid(jnp.dot(WzT_ref[...], mT) + jnp.dot(UzT_ref[...], xT)
                       + bzT_ref[...])
    r = jax.nn.sigmoid(jnp.dot(WrT_ref[...], mT) + jnp.dot(UrT_ref[...], xT)
                       + brT_ref[...])
    h_til = jnp.tanh(jnp.dot(WhT_ref[...], mT)
                     + jnp.dot(UhT_ref[...], r * xT) + bhT_ref[...])
    hT = (1.0 - z) * xT + z * h_til

    logits = jnp.dot(WoT_ref[...], hT) + boT_ref[...]
    lmax = jnp.max(logits, axis=0, keepdims=True)
    shifted = logits - lmax
    lse = jnp.log(jnp.sum(jnp.exp(shifted), axis=0, keepdims=True))
    out_ref[0] = shifted - lse


@jax.jit
def kernel(x_padded, x_lengths, edges, fingers, W_edge, Wz, Uz, bz,
           Wr, Ur, br, Wh, Uh, bh, W_out, b_out):
    del x_lengths, fingers
    grid = (B,)

    full = lambda b: (0, 0)
    outT = pl.pallas_call(
        _ggnn_kernel,
        grid=grid,
        in_specs=[
            pl.BlockSpec((1, D, N), lambda b: (b, 0, 0)),
            pl.BlockSpec((1, E, N, N), lambda b: (b, 0, 0, 0)),
            pl.BlockSpec((E, D, D), lambda b: (0, 0, 0)),
            pl.BlockSpec((D, D), full),
            pl.BlockSpec((D, D), full),
            pl.BlockSpec((D, 1), full),
            pl.BlockSpec((D, D), full),
            pl.BlockSpec((D, D), full),
            pl.BlockSpec((D, 1), full),
            pl.BlockSpec((D, D), full),
            pl.BlockSpec((D, D), full),
            pl.BlockSpec((D, 1), full),
            pl.BlockSpec((5, D), full),
            pl.BlockSpec((5, 1), full),
        ],
        out_specs=pl.BlockSpec((1, 5, N), lambda b: (b, 0, 0)),
        out_shape=jax.ShapeDtypeStruct((B, 5, N), jnp.float32),
    )(x_padded.transpose(0, 2, 1), edges,
      W_edge.transpose(0, 2, 1),
      Wz.T, Uz.T, bz.reshape(D, 1),
      Wr.T, Ur.T, br.reshape(D, 1),
      Wh.T, Uh.T, bh.reshape(D, 1),
      W_out.T, b_out.reshape(5, 1))
    return outT.transpose(0, 2, 1)
